# Initial kernel scaffold; baseline (speedup 1.0000x reference)
import jax
import jax.numpy as jnp
from jax.experimental import pallas as pl


def kernel(xyz1, xyz2, points1, points2, conv_w, conv_b):
    raise NotImplementedError("write your pallas kernel here")


# speedup vs baseline: 10.3785x; 10.3785x over previous
import functools

import jax
import jax.numpy as jnp
from jax import lax
from jax.experimental import pallas as pl
from jax.experimental.pallas import tpu as pltpu
from jax.experimental.pallas import tpu_sc as plsc

_B, _N, _S, _D1, _D2, _K = 4, 4096, 4096, 256, 256, 16
_L = 16
_NC, _NS = 2, 16
_NW = _NC * _NS
_RPW = _B * _N // _NW
_WPB = _NW // _B
_NITER = _N // _L


def _rne_bf16(v):
    u = lax.bitcast_convert_type(v, jnp.uint32)
    r = (u + 0x7FFF + ((u >> 16) & 1)) & jnp.uint32(0xFFFF0000)
    return lax.bitcast_convert_type(r, jnp.float32)


def _sc_interp_body(x1_hbm, x2_hbm, p2_hbm, out_hbm,
                    xv, yv, zv, xb, yb, zb, nv, qx, qy, qz, rows, orow, sem):
    cid = lax.axis_index("c")
    sid = lax.axis_index("s")
    wid = sid * _NC + cid
    b = wid // _WPB
    base = wid * _RPW
    cbase = (wid % _WPB) * _RPW

    xoff = (b * 3) * _N
    pltpu.sync_copy(x1_hbm.at[pl.ds(xoff, _N)], xv)
    pltpu.sync_copy(x1_hbm.at[pl.ds(xoff + _N, _N)], yv)
    pltpu.sync_copy(x1_hbm.at[pl.ds(xoff + 2 * _N, _N)], zv)
    qoff = (b * 3) * _N + cbase
    pltpu.sync_copy(x2_hbm.at[pl.ds(qoff, _RPW)], qx.at[pl.ds(0, _RPW)])
    pltpu.sync_copy(x2_hbm.at[pl.ds(qoff + _N, _RPW)], qy.at[pl.ds(0, _RPW)])
    pltpu.sync_copy(x2_hbm.at[pl.ds(qoff + 2 * _N, _RPW)], qz.at[pl.ds(0, _RPW)])

    def _norm_body(i, _):
        s = pl.ds(i * _L, _L)
        a, c, d = xv[s], yv[s], zv[s]
        nv[s] = (a * a + c * c) + d * d
        xb[s] = _rne_bf16(a)
        yb[s] = _rne_bf16(c)
        zb[s] = _rne_bf16(d)
        return 0
    lax.fori_loop(0, _NITER, _norm_body, 0)

    def _row_body(r, _):
        qxs = qx[pl.ds(r, _L)][0]
        qys = qy[pl.ds(r, _L)][0]
        qzs = qz[pl.ds(r, _L)][0]
        bx = _rne_bf16(jnp.full((_L,), qxs, jnp.float32))
        by = _rne_bf16(jnp.full((_L,), qys, jnp.float32))
        bz = _rne_bf16(jnp.full((_L,), qzs, jnp.float32))
        qn = (qxs * qxs + qys * qys) + qzs * qzs
        qnv = jnp.full((_L,), qn, jnp.float32)

        topd0 = jnp.full((_L,), jnp.inf, jnp.float32)
        topi0 = jnp.zeros((_L,), jnp.int32)

        def _cand_body(i, carry):
            topd, topi = carry
            s = pl.ds(i * _L, _L)
            dot = (bx * xb[s] + by * yb[s]) + bz * zb[s]
            key = (jnp.float32(-2.0) * dot + qnv) + nv[s]
            iv = lax.iota(jnp.int32, _L) + i * _L
            cd, ci = plsc.sort_key_val(key, iv, descending=True)
            m = topd <= cd
            nd = jnp.where(m, topd, cd)
            ni = jnp.where(m, topi, ci)
            topd, topi = plsc.sort_key_val(nd, ni, descending=False)
            return topd, topi

        topd, topi = lax.fori_loop(0, _NITER, _cand_body, (topd0, topi0))

        d = topd
        dr = 1.0 / (d + 1e-8)
        norm = jnp.sum(dr)
        w = dr / jnp.full((_L,), norm, jnp.float32)

        gidx = topi + b * _S
        pltpu.async_copy(p2_hbm.at[gidx], rows, sem).wait()

        wb = [jnp.full((_L,), w[j], jnp.float32) for j in range(_K)]
        for c in range(_D2 // _L):
            sc = pl.ds(c * _L, _L)
            acc = jnp.zeros((_L,), jnp.float32)
            for j in range(_K):
                acc = acc + wb[j] * rows[j, sc]
            orow[sc] = acc
        pltpu.sync_copy(orow, out_hbm.at[pl.ds((base + r) * _D2, _D2)])
        return 0

    lax.fori_loop(0, _RPW, _row_body, 0)


def _sc_interp(x1f, x2f, p2f):
    mesh = plsc.VectorSubcoreMesh(core_axis_name="c", subcore_axis_name="s")
    return pl.kernel(
        _sc_interp_body,
        out_type=jax.ShapeDtypeStruct((_B * _N * _D2,), jnp.float32),
        mesh=mesh,
        compiler_params=pltpu.CompilerParams(needs_layout_passes=False),
        scratch_types=[
            pltpu.VMEM((_N,), jnp.float32),
            pltpu.VMEM((_N,), jnp.float32),
            pltpu.VMEM((_N,), jnp.float32),
            pltpu.VMEM((_N,), jnp.float32),
            pltpu.VMEM((_N,), jnp.float32),
            pltpu.VMEM((_N,), jnp.float32),
            pltpu.VMEM((_N,), jnp.float32),
            pltpu.VMEM((_RPW + _L,), jnp.float32),
            pltpu.VMEM((_RPW + _L,), jnp.float32),
            pltpu.VMEM((_RPW + _L,), jnp.float32),
            pltpu.VMEM((_K, _D2), jnp.float32),
            pltpu.VMEM((_D2,), jnp.float32),
            pltpu.SemaphoreType.DMA,
        ],
    )(x1f, x2f, p2f)


def _conv_body(p1_ref, it_ref, w_ref, b_ref, o_ref):
    dn = (((1,), (1,)), ((), ()))
    wl = w_ref[:, : _D1].astype(jnp.bfloat16)
    wr = w_ref[:, _D1:].astype(jnp.bfloat16)
    o_ref[...] = (
        lax.dot_general(p1_ref[...].astype(jnp.bfloat16), wl, dn,
                        preferred_element_type=jnp.float32)
        + lax.dot_general(it_ref[...].astype(jnp.bfloat16), wr, dn,
                          preferred_element_type=jnp.float32)
        + b_ref[...]
    )


def _conv(p1f, interp, conv_w, bias2d):
    rows_blk = 512
    grid = (_B * _N // rows_blk,)
    return pl.pallas_call(
        _conv_body,
        out_shape=jax.ShapeDtypeStruct((_B * _N, _D2), jnp.float32),
        grid=grid,
        in_specs=[
            pl.BlockSpec((rows_blk, _D1), lambda i: (i, 0)),
            pl.BlockSpec((rows_blk, _D2), lambda i: (i, 0)),
            pl.BlockSpec((_D2, _D1 + _D2), lambda i: (0, 0)),
            pl.BlockSpec((1, _D2), lambda i: (0, 0)),
        ],
        out_specs=pl.BlockSpec((rows_blk, _D2), lambda i: (i, 0)),
    )(p1f, interp, conv_w, bias2d)


def kernel(xyz1, xyz2, points1, points2, conv_w, conv_b):
    x1f = xyz1.transpose(0, 2, 1).reshape(-1)
    x2f = xyz2.transpose(0, 2, 1).reshape(-1)
    p2f = points2.reshape(_B * _S, _D2)
    interp = _sc_interp(x1f, x2f, p2f).reshape(_B * _N, _D2)

    p1f = points1.reshape(_B * _N, _D1)
    bias2d = conv_b.reshape(1, _D2)
    out = _conv(p1f, interp, conv_w, bias2d)
    return out.reshape(_B, _N, _D2)


# speedup vs baseline: 19.0697x; 1.8374x over previous
import functools

import jax
import jax.numpy as jnp
from jax import lax
from jax.experimental import pallas as pl
from jax.experimental.pallas import tpu as pltpu
from jax.experimental.pallas import tpu_sc as plsc

_B, _N, _S, _D1, _D2, _K = 4, 4096, 4096, 256, 256, 16
_L = 16
_NC, _NS = 2, 16
_NW = _NC * _NS
_RPW = _B * _N // _NW
_WPB = _NW // _B
_NITER = _N // _L
_IL = 4


def _rne_bf16(v):
    u = lax.bitcast_convert_type(v, jnp.uint32)
    r = (u + 0x7FFF + ((u >> 16) & 1)) & jnp.uint32(0xFFFF0000)
    return lax.bitcast_convert_type(r, jnp.float32)


def _sc_interp_body(x1_hbm, x2_hbm, p2_hbm, out_hbm,
                    xv, yv, zv, xb, yb, zb, nv, qx, qy, qz, rows, orow, sem):
    cid = lax.axis_index("c")
    sid = lax.axis_index("s")
    wid = sid * _NC + cid
    b = wid // _WPB
    base = wid * _RPW
    cbase = (wid % _WPB) * _RPW

    xoff = (b * 3) * _N
    pltpu.sync_copy(x1_hbm.at[pl.ds(xoff, _N)], xv)
    pltpu.sync_copy(x1_hbm.at[pl.ds(xoff + _N, _N)], yv)
    pltpu.sync_copy(x1_hbm.at[pl.ds(xoff + 2 * _N, _N)], zv)
    qoff = (b * 3) * _N + cbase
    pltpu.sync_copy(x2_hbm.at[pl.ds(qoff, _RPW)], qx.at[pl.ds(0, _RPW)])
    pltpu.sync_copy(x2_hbm.at[pl.ds(qoff + _N, _RPW)], qy.at[pl.ds(0, _RPW)])
    pltpu.sync_copy(x2_hbm.at[pl.ds(qoff + 2 * _N, _RPW)], qz.at[pl.ds(0, _RPW)])

    def _norm_body(i, _):
        s = pl.ds(i * _L, _L)
        a, c, d = xv[s], yv[s], zv[s]
        nv[s] = (a * a + c * c) + d * d
        xb[s] = _rne_bf16(a)
        yb[s] = _rne_bf16(c)
        zb[s] = _rne_bf16(d)
        return 0
    lax.fori_loop(0, _NITER, _norm_body, 0)

    def _group_body(g, _):
        r0 = g * _IL
        bx2, by2, bz2, qnv = [], [], [], []
        for u in range(_IL):
            qxs = qx[pl.ds(r0 + u, _L)][0]
            qys = qy[pl.ds(r0 + u, _L)][0]
            qzs = qz[pl.ds(r0 + u, _L)][0]
            bx2.append(_rne_bf16(jnp.full((_L,), qxs, jnp.float32)) * -2.0)
            by2.append(_rne_bf16(jnp.full((_L,), qys, jnp.float32)) * -2.0)
            bz2.append(_rne_bf16(jnp.full((_L,), qzs, jnp.float32)) * -2.0)
            qn = (qxs * qxs + qys * qys) + qzs * qzs
            qnv.append(jnp.full((_L,), qn, jnp.float32))

        init = (tuple(jnp.full((_L,), jnp.inf, jnp.float32) for _ in range(_IL))
                + tuple(jnp.zeros((_L,), jnp.int32) for _ in range(_IL)))

        def _cand_body(i, carry):
            topd = list(carry[:_IL])
            topi = list(carry[_IL:])
            s = pl.ds(i * _L, _L)
            xs, ys, zs, ns = xb[s], yb[s], zb[s], nv[s]
            iv = lax.iota(jnp.int32, _L) + i * _L
            for u in range(_IL):
                t = (bx2[u] * xs + by2[u] * ys) + bz2[u] * zs
                key = (t + qnv[u]) + ns
                cd, ci = plsc.sort_key_val(key, iv, descending=True)
                m = topd[u] <= cd
                nd = jnp.where(m, topd[u], cd)
                ni = jnp.where(m, topi[u], ci)
                topd[u], topi[u] = plsc.sort_key_val(nd, ni, descending=False)
            return tuple(topd) + tuple(topi)

        fin = lax.fori_loop(0, _NITER, _cand_body, init)

        for u in range(_IL):
            d = fin[u]
            topi = fin[_IL + u]
            dr = 1.0 / (d + 1e-8)
            norm = jnp.sum(dr)
            w = dr / jnp.full((_L,), norm, jnp.float32)

            gidx = topi + b * _S
            pltpu.async_copy(p2_hbm.at[gidx], rows, sem).wait()

            wbv = [jnp.full((_L,), w[j], jnp.float32) for j in range(_K)]
            for c in range(_D2 // _L):
                sc = pl.ds(c * _L, _L)
                acc = jnp.zeros((_L,), jnp.float32)
                for j in range(_K):
                    acc = acc + wbv[j] * rows[j, sc]
                orow[sc] = acc
            pltpu.sync_copy(orow, out_hbm.at[pl.ds((base + r0 + u) * _D2, _D2)])
        return 0

    lax.fori_loop(0, _RPW // _IL, _group_body, 0)


def _sc_interp(x1f, x2f, p2f):
    mesh = plsc.VectorSubcoreMesh(core_axis_name="c", subcore_axis_name="s")
    return pl.kernel(
        _sc_interp_body,
        out_type=jax.ShapeDtypeStruct((_B * _N * _D2,), jnp.float32),
        mesh=mesh,
        compiler_params=pltpu.CompilerParams(needs_layout_passes=False),
        scratch_types=[
            pltpu.VMEM((_N,), jnp.float32),
            pltpu.VMEM((_N,), jnp.float32),
            pltpu.VMEM((_N,), jnp.float32),
            pltpu.VMEM((_N,), jnp.float32),
            pltpu.VMEM((_N,), jnp.float32),
            pltpu.VMEM((_N,), jnp.float32),
            pltpu.VMEM((_N,), jnp.float32),
            pltpu.VMEM((_RPW + _L,), jnp.float32),
            pltpu.VMEM((_RPW + _L,), jnp.float32),
            pltpu.VMEM((_RPW + _L,), jnp.float32),
            pltpu.VMEM((_K, _D2), jnp.float32),
            pltpu.VMEM((_D2,), jnp.float32),
            pltpu.SemaphoreType.DMA,
        ],
    )(x1f, x2f, p2f)


def _conv_body(p1_ref, it_ref, w_ref, b_ref, o_ref):
    dn = (((1,), (1,)), ((), ()))
    wl = w_ref[:, : _D1].astype(jnp.bfloat16)
    wr = w_ref[:, _D1:].astype(jnp.bfloat16)
    o_ref[...] = (
        lax.dot_general(p1_ref[...].astype(jnp.bfloat16), wl, dn,
                        preferred_element_type=jnp.float32)
        + lax.dot_general(it_ref[...].astype(jnp.bfloat16), wr, dn,
                          preferred_element_type=jnp.float32)
        + b_ref[...]
    )


def _conv(p1f, interp, conv_w, bias2d):
    rows_blk = 512
    grid = (_B * _N // rows_blk,)
    return pl.pallas_call(
        _conv_body,
        out_shape=jax.ShapeDtypeStruct((_B * _N, _D2), jnp.float32),
        grid=grid,
        in_specs=[
            pl.BlockSpec((rows_blk, _D1), lambda i: (i, 0)),
            pl.BlockSpec((rows_blk, _D2), lambda i: (i, 0)),
            pl.BlockSpec((_D2, _D1 + _D2), lambda i: (0, 0)),
            pl.BlockSpec((1, _D2), lambda i: (0, 0)),
        ],
        out_specs=pl.BlockSpec((rows_blk, _D2), lambda i: (i, 0)),
    )(p1f, interp, conv_w, bias2d)


def kernel(xyz1, xyz2, points1, points2, conv_w, conv_b):
    x1f = xyz1.transpose(0, 2, 1).reshape(-1)
    x2f = xyz2.transpose(0, 2, 1).reshape(-1)
    p2f = points2.reshape(_B * _S, _D2)
    interp = _sc_interp(x1f, x2f, p2f).reshape(_B * _N, _D2)

    p1f = points1.reshape(_B * _N, _D1)
    bias2d = conv_b.reshape(1, _D2)
    out = _conv(p1f, interp, conv_w, bias2d)
    return out.reshape(_B, _N, _D2)


# speedup vs baseline: 26.4577x; 1.3874x over previous
import functools

import jax
import jax.numpy as jnp
from jax import lax
from jax.experimental import pallas as pl
from jax.experimental.pallas import tpu as pltpu
from jax.experimental.pallas import tpu_sc as plsc

_B, _N, _S, _D1, _D2, _K = 4, 4096, 4096, 256, 256, 16
_L = 16
_NC, _NS = 2, 16
_NW = _NC * _NS
_RPW = _B * _N // _NW
_WPB = _NW // _B
_NITER = _N // _L
_IL = 4


def _rne_bf16(v):
    u = lax.bitcast_convert_type(v, jnp.uint32)
    r = (u + 0x7FFF + ((u >> 16) & 1)) & jnp.uint32(0xFFFF0000)
    return lax.bitcast_convert_type(r, jnp.float32)


def _sc_interp_body(x1_hbm, x2_hbm, p2_hbm, out_hbm,
                    xv, yv, zv, xb, yb, zb, nv, qx, qy, qz, rows, orow,
                    sem, sem2):
    cid = lax.axis_index("c")
    sid = lax.axis_index("s")
    wid = sid * _NC + cid
    b = wid // _WPB
    base = wid * _RPW
    cbase = (wid % _WPB) * _RPW

    xoff = (b * 3) * _N
    pltpu.sync_copy(x1_hbm.at[pl.ds(xoff, _N)], xv)
    pltpu.sync_copy(x1_hbm.at[pl.ds(xoff + _N, _N)], yv)
    pltpu.sync_copy(x1_hbm.at[pl.ds(xoff + 2 * _N, _N)], zv)
    qoff = (b * 3) * _N + cbase
    pltpu.sync_copy(x2_hbm.at[pl.ds(qoff, _RPW)], qx.at[pl.ds(0, _RPW)])
    pltpu.sync_copy(x2_hbm.at[pl.ds(qoff + _N, _RPW)], qy.at[pl.ds(0, _RPW)])
    pltpu.sync_copy(x2_hbm.at[pl.ds(qoff + 2 * _N, _RPW)], qz.at[pl.ds(0, _RPW)])

    def _norm_body(i, _):
        s = pl.ds(i * _L, _L)
        a, c, d = xv[s], yv[s], zv[s]
        nv[s] = (a * a + c * c) + d * d
        xb[s] = _rne_bf16(a)
        yb[s] = _rne_bf16(c)
        zb[s] = _rne_bf16(d)
        return 0
    lax.fori_loop(0, _NITER, _norm_body, 0)

    def _group_body(g, _):
        r0 = g * _IL
        bx2, by2, bz2, qnv = [], [], [], []
        for u in range(_IL):
            qxs = qx[pl.ds(r0 + u, _L)][0]
            qys = qy[pl.ds(r0 + u, _L)][0]
            qzs = qz[pl.ds(r0 + u, _L)][0]
            bx2.append(_rne_bf16(jnp.full((_L,), qxs, jnp.float32)) * -2.0)
            by2.append(_rne_bf16(jnp.full((_L,), qys, jnp.float32)) * -2.0)
            bz2.append(_rne_bf16(jnp.full((_L,), qzs, jnp.float32)) * -2.0)
            qn = (qxs * qxs + qys * qys) + qzs * qzs
            qnv.append(jnp.full((_L,), qn, jnp.float32))

        init = (tuple(jnp.full((_L,), jnp.inf, jnp.float32) for _ in range(_IL))
                + tuple(jnp.zeros((_L,), jnp.int32) for _ in range(_IL)))

        def _cand_body(i, carry):
            topd = list(carry[:_IL])
            topi = list(carry[_IL:])
            s = pl.ds(i * _L, _L)
            xs, ys, zs, ns = xb[s], yb[s], zb[s], nv[s]
            iv = lax.iota(jnp.int32, _L) + i * _L
            for u in range(_IL):
                t = (bx2[u] * xs + by2[u] * ys) + bz2[u] * zs
                key = (t + qnv[u]) + ns
                cd, ci = plsc.sort_key_val(key, iv, descending=True)
                m = topd[u] <= cd
                nd = jnp.where(m, topd[u], cd)
                ni = jnp.where(m, topi[u], ci)
                topd[u], topi[u] = plsc.sort_key_val(nd, ni, descending=False)
            return tuple(topd) + tuple(topi)

        fin = lax.fori_loop(0, _NITER, _cand_body, init)

        gcopies = []
        for u in range(_IL):
            gidx = fin[_IL + u] + b * _S
            gcopies.append(pltpu.async_copy(p2_hbm.at[gidx], rows.at[u], sem))

        ocopies = []
        for u in range(_IL):
            d = fin[u]
            dr = 1.0 / (d + 1e-8)
            norm = jnp.sum(dr)
            w = dr / jnp.full((_L,), norm, jnp.float32)
            wbv = [jnp.full((_L,), w[j], jnp.float32) for j in range(_K)]
            gcopies[u].wait()
            for c in range(_D2 // _L):
                sc = pl.ds(c * _L, _L)
                acc = jnp.zeros((_L,), jnp.float32)
                for j in range(_K):
                    acc = acc + wbv[j] * rows[u, j, sc]
                orow[u, sc] = acc
            ocopies.append(pltpu.async_copy(
                orow.at[u], out_hbm.at[pl.ds((base + r0 + u) * _D2, _D2)], sem2))
        for cpy in ocopies:
            cpy.wait()
        return 0

    lax.fori_loop(0, _RPW // _IL, _group_body, 0)


def _sc_interp(x1f, x2f, p2f):
    mesh = plsc.VectorSubcoreMesh(core_axis_name="c", subcore_axis_name="s")
    return pl.kernel(
        _sc_interp_body,
        out_type=jax.ShapeDtypeStruct((_B * _N * _D2,), jnp.float32),
        mesh=mesh,
        compiler_params=pltpu.CompilerParams(needs_layout_passes=False),
        scratch_types=[
            pltpu.VMEM((_N,), jnp.float32),
            pltpu.VMEM((_N,), jnp.float32),
            pltpu.VMEM((_N,), jnp.float32),
            pltpu.VMEM((_N,), jnp.float32),
            pltpu.VMEM((_N,), jnp.float32),
            pltpu.VMEM((_N,), jnp.float32),
            pltpu.VMEM((_N,), jnp.float32),
            pltpu.VMEM((_RPW + _L,), jnp.float32),
            pltpu.VMEM((_RPW + _L,), jnp.float32),
            pltpu.VMEM((_RPW + _L,), jnp.float32),
            pltpu.VMEM((_IL, _K, _D2), jnp.float32),
            pltpu.VMEM((_IL, _D2), jnp.float32),
            pltpu.SemaphoreType.DMA,
            pltpu.SemaphoreType.DMA,
        ],
    )(x1f, x2f, p2f)


def _conv_body(p1_ref, it_ref, w_ref, b_ref, o_ref):
    dn = (((1,), (1,)), ((), ()))
    wl = w_ref[:, : _D1].astype(jnp.bfloat16)
    wr = w_ref[:, _D1:].astype(jnp.bfloat16)
    o_ref[...] = (
        lax.dot_general(p1_ref[...].astype(jnp.bfloat16), wl, dn,
                        preferred_element_type=jnp.float32)
        + lax.dot_general(it_ref[...].astype(jnp.bfloat16), wr, dn,
                          preferred_element_type=jnp.float32)
        + b_ref[...]
    )


def _conv(p1f, interp, conv_w, bias2d):
    rows_blk = 512
    grid = (_B * _N // rows_blk,)
    return pl.pallas_call(
        _conv_body,
        out_shape=jax.ShapeDtypeStruct((_B * _N, _D2), jnp.float32),
        grid=grid,
        in_specs=[
            pl.BlockSpec((rows_blk, _D1), lambda i: (i, 0)),
            pl.BlockSpec((rows_blk, _D2), lambda i: (i, 0)),
            pl.BlockSpec((_D2, _D1 + _D2), lambda i: (0, 0)),
            pl.BlockSpec((1, _D2), lambda i: (0, 0)),
        ],
        out_specs=pl.BlockSpec((rows_blk, _D2), lambda i: (i, 0)),
    )(p1f, interp, conv_w, bias2d)


def kernel(xyz1, xyz2, points1, points2, conv_w, conv_b):
    x1f = xyz1.transpose(0, 2, 1).reshape(-1)
    x2f = xyz2.transpose(0, 2, 1).reshape(-1)
    p2f = points2.reshape(_B * _S, _D2)
    interp = _sc_interp(x1f, x2f, p2f).reshape(_B * _N, _D2)

    p1f = points1.reshape(_B * _N, _D1)
    bias2d = conv_b.reshape(1, _D2)
    out = _conv(p1f, interp, conv_w, bias2d)
    return out.reshape(_B, _N, _D2)
